# baseline (speedup 1.0000x reference)
import functools

import jax
import jax.numpy as jnp
from jax import lax
from jax.experimental import pallas as pl
from jax.experimental.pallas import tpu as pltpu
from jax.experimental.pallas import tpu_sc as plsc

B = 16384
D = 16
NC = 2
NS = 16
NW = NC * NS
K = 8
SLAB = 128
CHUNKS = 2


def _make_gather_body(n):
    bpw = n // NW
    ngrp = bpw // K
    nbody = (ngrp - 2) // 6
    tail = ngrp - 6 * nbody

    def body_fn(uidx_hbm, iidx_hbm, utab_hbm, itab_hbm, uout_hbm, iout_hbm,
                uidx_v, iidx_v, urows_v, irows_v, uslab, islab,
                usem0, usem1, usem2, isem0, isem1, isem2):
        wid = lax.axis_index("s") * NC + lax.axis_index("c")
        base = wid * bpw
        pltpu.sync_copy(uidx_hbm.at[pl.ds(base, bpw)], uidx_v)
        pltpu.sync_copy(iidx_hbm.at[pl.ds(base, bpw)], iidx_v)

        rows16 = lax.iota(jnp.int32, 16)
        usems = (usem0, usem1, usem2)
        isems = (isem0, isem1, isem2)

        def fire_grp(tv, half, buf):
            uvec = uidx_v[pl.ds(tv * 16, 16)]
            ivec = iidx_v[pl.ds(tv * 16, 16)]
            for j in range(K):
                v = uvec[half + j]
                off = pl.multiple_of((v // SLAB) * SLAB, SLAB)
                pltpu.async_copy(utab_hbm.at[:, pl.ds(off, SLAB)],
                                 uslab.at[buf, j], usems[buf])
                w = ivec[half + j]
                offi = pl.multiple_of((w // SLAB) * SLAB, SLAB)
                pltpu.async_copy(itab_hbm.at[:, pl.ds(offi, SLAB)],
                                 islab.at[buf, j], isems[buf])

        def drain_grp(tv, half, buf):
            uvec = uidx_v[pl.ds(tv * 16, 16)]
            ivec = iidx_v[pl.ds(tv * 16, 16)]
            ibase = tv * 16 + half
            for j in range(K):
                col = jnp.full((16,), ibase + j, jnp.int32)
                pltpu.make_async_copy(utab_hbm.at[:, pl.ds(0, SLAB)],
                                      uslab.at[buf, j], usems[buf]).wait()
                v = uvec[half + j]
                lane = jnp.full((16,), v % SLAB, jnp.int32)
                emb = plsc.load_gather(uslab.at[buf, j], [rows16, lane])
                plsc.store_scatter(urows_v, [rows16, col], emb)
                pltpu.make_async_copy(itab_hbm.at[:, pl.ds(0, SLAB)],
                                      islab.at[buf, j], isems[buf]).wait()
                w = ivec[half + j]
                lanei = jnp.full((16,), w % SLAB, jnp.int32)
                embi = plsc.load_gather(islab.at[buf, j], [rows16, lanei])
                plsc.store_scatter(irows_v, [rows16, col], embi)

        fire_grp(0, 0, 0)
        fire_grp(0, 8, 1)

        def body(s, carry):
            for k in range(6):
                fire_grp(3 * s + (k + 2) // 2, ((k + 2) % 2) * 8,
                         (k + 2) % 3)
                drain_grp(3 * s + k // 2, (k % 2) * 8, k % 3)
            return carry

        lax.fori_loop(0, nbody, body, 0)

        for k in range(tail):
            h = 6 * nbody + k
            if k < tail - 2:
                hf = h + 2
                fire_grp(hf // 2, (hf % 2) * 8, hf % 3)
            drain_grp(h // 2, (h % 2) * 8, h % 3)

        pltpu.sync_copy(urows_v, uout_hbm.at[:, pl.ds(base, bpw)])
        pltpu.sync_copy(irows_v, iout_hbm.at[:, pl.ds(base, bpw)])

    return body_fn


@functools.cache
def _sc_gather(n):
    bpw = n // NW
    return functools.partial(
        pl.kernel,
        out_type=(
            jax.ShapeDtypeStruct((D, n), jnp.float32),
            jax.ShapeDtypeStruct((D, n), jnp.float32),
        ),
        mesh=plsc.VectorSubcoreMesh(
            core_axis_name="c", subcore_axis_name="s", num_cores=NC,
            num_subcores=NS),
        scratch_types=[
            pltpu.VMEM((bpw,), jnp.int32),
            pltpu.VMEM((bpw,), jnp.int32),
            pltpu.VMEM((D, bpw), jnp.float32),
            pltpu.VMEM((D, bpw), jnp.float32),
            pltpu.VMEM((3, K, D, SLAB), jnp.float32),
            pltpu.VMEM((3, K, D, SLAB), jnp.float32),
            pltpu.SemaphoreType.DMA,
            pltpu.SemaphoreType.DMA,
            pltpu.SemaphoreType.DMA,
            pltpu.SemaphoreType.DMA,
            pltpu.SemaphoreType.DMA,
            pltpu.SemaphoreType.DMA,
        ],
        compiler_params=pltpu.CompilerParams(needs_layout_passes=False),
    )(_make_gather_body(n))


BM = 2048


def _mlp_body(ue_ref, ie_ref, w1u_ref, w1i_ref, b1_ref, w2_ref, b2_ref,
              w3_ref, b3_ref, out_ref):
    h1 = jnp.dot(w1u_ref[...], ue_ref[...],
                 preferred_element_type=jnp.float32)
    h1 += jnp.dot(w1i_ref[...], ie_ref[...],
                  preferred_element_type=jnp.float32)
    h1 = jnp.maximum(h1 + b1_ref[...], 0.0)
    h2 = jnp.maximum(
        jnp.dot(w2_ref[...], h1, preferred_element_type=jnp.float32)
        + b2_ref[...], 0.0)
    out_ref[...] = (
        jnp.dot(w3_ref[...], h2, preferred_element_type=jnp.float32)
        + b3_ref[...])


def _full(shape):
    return pl.BlockSpec(shape, lambda i: (0,) * len(shape))


@functools.cache
def _mlp(n):
    return pl.pallas_call(
        _mlp_body,
        grid=(n // BM,),
        in_specs=[
            pl.BlockSpec((D, BM), lambda i: (0, i)),
            pl.BlockSpec((D, BM), lambda i: (0, i)),
            _full((64, D)),
            _full((64, D)),
            _full((64, 1)),
            _full((32, 64)),
            _full((32, 1)),
            _full((1, 32)),
            _full((1, 1)),
        ],
        out_specs=pl.BlockSpec((1, BM), lambda i: (0, i)),
        out_shape=jax.ShapeDtypeStruct((1, n), jnp.float32),
    )


def kernel(user, item, user_table, item_table, W1, b1, W2, b2, W3, b3):
    uidx = user.astype(jnp.int32)
    iidx = item.astype(jnp.int32)
    utT = user_table.T
    itT = item_table.T
    w1u = W1[:, :D]
    w1i = W1[:, D:]
    b1c = b1.reshape(64, 1)
    b2c = b2.reshape(32, 1)
    b3c = b3.reshape(1, 1)

    h = B // CHUNKS
    embs = [_sc_gather(h)(uidx[c * h:(c + 1) * h], iidx[c * h:(c + 1) * h],
                          utT, itT) for c in range(CHUNKS)]
    outs = [_mlp(h)(ue, ie, w1u, w1i, b1c, W2, b2c, W3, b3c)
            for ue, ie in embs]
    return jnp.concatenate(outs, axis=1).reshape(B, 1)


# speedup vs baseline: 1.0936x; 1.0936x over previous
import functools

import jax
import jax.numpy as jnp
from jax import lax
from jax.experimental import pallas as pl
from jax.experimental.pallas import tpu as pltpu
from jax.experimental.pallas import tpu_sc as plsc

B = 16384
D = 16
NC = 2
NS = 16
NW = NC * NS
K = 8
SLAB = 128
CHUNKS = 1


def _make_gather_body(n):
    bpw = n // NW
    ngrp = bpw // K
    nbody = (ngrp - 2) // 6
    tail = ngrp - 6 * nbody

    def body_fn(uidx_hbm, iidx_hbm, utab_hbm, itab_hbm, uout_hbm, iout_hbm,
                uidx_v, iidx_v, urows_v, irows_v, uslab, islab,
                usem0, usem1, usem2, isem0, isem1, isem2):
        wid = lax.axis_index("s") * NC + lax.axis_index("c")
        base = wid * bpw
        pltpu.sync_copy(uidx_hbm.at[pl.ds(base, bpw)], uidx_v)
        pltpu.sync_copy(iidx_hbm.at[pl.ds(base, bpw)], iidx_v)

        rows16 = lax.iota(jnp.int32, 16)
        usems = (usem0, usem1, usem2)
        isems = (isem0, isem1, isem2)

        def fire_grp(tv, half, buf):
            uvec = uidx_v[pl.ds(tv * 16, 16)]
            ivec = iidx_v[pl.ds(tv * 16, 16)]
            for j in range(K):
                v = uvec[half + j]
                off = pl.multiple_of((v // SLAB) * SLAB, SLAB)
                pltpu.async_copy(utab_hbm.at[:, pl.ds(off, SLAB)],
                                 uslab.at[buf, j], usems[buf])
                w = ivec[half + j]
                offi = pl.multiple_of((w // SLAB) * SLAB, SLAB)
                pltpu.async_copy(itab_hbm.at[:, pl.ds(offi, SLAB)],
                                 islab.at[buf, j], isems[buf])

        def drain_grp(tv, half, buf):
            uvec = uidx_v[pl.ds(tv * 16, 16)]
            ivec = iidx_v[pl.ds(tv * 16, 16)]
            ibase = tv * 16 + half
            for j in range(K):
                col = jnp.full((16,), ibase + j, jnp.int32)
                pltpu.make_async_copy(utab_hbm.at[:, pl.ds(0, SLAB)],
                                      uslab.at[buf, j], usems[buf]).wait()
                v = uvec[half + j]
                lane = jnp.full((16,), v % SLAB, jnp.int32)
                emb = plsc.load_gather(uslab.at[buf, j], [rows16, lane])
                plsc.store_scatter(urows_v, [rows16, col], emb)
                pltpu.make_async_copy(itab_hbm.at[:, pl.ds(0, SLAB)],
                                      islab.at[buf, j], isems[buf]).wait()
                w = ivec[half + j]
                lanei = jnp.full((16,), w % SLAB, jnp.int32)
                embi = plsc.load_gather(islab.at[buf, j], [rows16, lanei])
                plsc.store_scatter(irows_v, [rows16, col], embi)

        fire_grp(0, 0, 0)
        fire_grp(0, 8, 1)

        def body(s, carry):
            for k in range(6):
                fire_grp(3 * s + (k + 2) // 2, ((k + 2) % 2) * 8,
                         (k + 2) % 3)
                drain_grp(3 * s + k // 2, (k % 2) * 8, k % 3)
            return carry

        lax.fori_loop(0, nbody, body, 0)

        for k in range(tail):
            h = 6 * nbody + k
            if k < tail - 2:
                hf = h + 2
                fire_grp(hf // 2, (hf % 2) * 8, hf % 3)
            drain_grp(h // 2, (h % 2) * 8, h % 3)

        pltpu.sync_copy(urows_v, uout_hbm.at[:, pl.ds(base, bpw)])
        pltpu.sync_copy(irows_v, iout_hbm.at[:, pl.ds(base, bpw)])

    return body_fn


@functools.cache
def _sc_gather(n):
    bpw = n // NW
    return functools.partial(
        pl.kernel,
        out_type=(
            jax.ShapeDtypeStruct((D, n), jnp.float32),
            jax.ShapeDtypeStruct((D, n), jnp.float32),
        ),
        mesh=plsc.VectorSubcoreMesh(
            core_axis_name="c", subcore_axis_name="s", num_cores=NC,
            num_subcores=NS),
        scratch_types=[
            pltpu.VMEM((bpw,), jnp.int32),
            pltpu.VMEM((bpw,), jnp.int32),
            pltpu.VMEM((D, bpw), jnp.float32),
            pltpu.VMEM((D, bpw), jnp.float32),
            pltpu.VMEM((3, K, D, SLAB), jnp.float32),
            pltpu.VMEM((3, K, D, SLAB), jnp.float32),
            pltpu.SemaphoreType.DMA,
            pltpu.SemaphoreType.DMA,
            pltpu.SemaphoreType.DMA,
            pltpu.SemaphoreType.DMA,
            pltpu.SemaphoreType.DMA,
            pltpu.SemaphoreType.DMA,
        ],
        compiler_params=pltpu.CompilerParams(needs_layout_passes=False),
    )(_make_gather_body(n))


BM = 16384


def _mlp_body(ue_ref, ie_ref, w1u_ref, w1i_ref, b1_ref, w2_ref, b2_ref,
              w3_ref, b3_ref, out_ref):
    h1 = jnp.dot(w1u_ref[...], ue_ref[...],
                 preferred_element_type=jnp.float32)
    h1 += jnp.dot(w1i_ref[...], ie_ref[...],
                  preferred_element_type=jnp.float32)
    h1 = jnp.maximum(h1 + b1_ref[...], 0.0)
    h2 = jnp.maximum(
        jnp.dot(w2_ref[...], h1, preferred_element_type=jnp.float32)
        + b2_ref[...], 0.0)
    out_ref[...] = (
        jnp.dot(w3_ref[...], h2, preferred_element_type=jnp.float32)
        + b3_ref[...])


def _full(shape):
    return pl.BlockSpec(shape, lambda i: (0,) * len(shape))


@functools.cache
def _mlp(n):
    return pl.pallas_call(
        _mlp_body,
        grid=(n // BM,),
        in_specs=[
            pl.BlockSpec((D, BM), lambda i: (0, i)),
            pl.BlockSpec((D, BM), lambda i: (0, i)),
            _full((64, D)),
            _full((64, D)),
            _full((64, 1)),
            _full((32, 64)),
            _full((32, 1)),
            _full((1, 32)),
            _full((1, 1)),
        ],
        out_specs=pl.BlockSpec((1, BM), lambda i: (0, i)),
        out_shape=jax.ShapeDtypeStruct((1, n), jnp.float32),
    )


def kernel(user, item, user_table, item_table, W1, b1, W2, b2, W3, b3):
    uidx = user.astype(jnp.int32)
    iidx = item.astype(jnp.int32)
    utT = user_table.T
    itT = item_table.T
    w1u = W1[:, :D]
    w1i = W1[:, D:]
    b1c = b1.reshape(64, 1)
    b2c = b2.reshape(32, 1)
    b3c = b3.reshape(1, 1)

    h = B // CHUNKS
    embs = [_sc_gather(h)(uidx[c * h:(c + 1) * h], iidx[c * h:(c + 1) * h],
                          utT, itT) for c in range(CHUNKS)]
    outs = [_mlp(h)(ue, ie, w1u, w1i, b1c, W2, b2c, W3, b3c)
            for ue, ie in embs]
    return jnp.concatenate(outs, axis=1).reshape(B, 1)
